# initial kernel scaffold (unmeasured)
import jax
import jax.numpy as jnp
from jax import lax
from jax.experimental import pallas as pl
from jax.experimental.pallas import tpu as pltpu

N_DEV = 8
N_PASS = 2


def kernel(x, w_mat, scale_x, scale_w):
    x = x.astype(jnp.bfloat16)
    w = w_mat.astype(jnp.bfloat16)
    m, k_sh = x.shape
    _, n = w.shape
    ring_rows = m // N_DEV
    blk = ring_rows // N_PASS
    n_hops = 2 * (N_DEV - 1)
    total_hops = N_PASS * n_hops

    def body(x_ref, w_ref, sx_ref, sw_ref, out_ref,
             comm, acc, send_sems, recv_sems, credit_sem, out_sem):
        d = lax.axis_index("i")
        left = (d - 1 + N_DEV) % N_DEV
        right = (d + 1) % N_DEV
        scale = sx_ref[0] * sw_ref[0]

        barrier = pltpu.get_barrier_semaphore()
        for nbr in (left, right):
            pl.semaphore_signal(barrier, inc=1, device_id=(nbr,),
                                device_id_type=pl.DeviceIdType.MESH)
        pl.semaphore_wait(barrier, 2)

        def partial(block, j):
            rows = x_ref[pl.ds(block * ring_rows + j * blk, blk), :]
            return jnp.dot(rows, w_ref[...], preferred_element_type=jnp.float32)

        def store_out(slot, block, j):
            cp = pltpu.make_async_copy(
                comm.at[slot],
                out_ref.at[pl.ds(block * ring_rows + j * blk, blk), :],
                out_sem,
            )
            cp.start()
            cp.wait()

        h = 0
        for j in range(N_PASS):
            comm[0] = partial(d, j)
            for k in range(n_hops):
                send_slot = k % 2
                recv_slot = (k + 1) % 2
                if h >= 1:
                    pl.semaphore_wait(credit_sem, 1)
                rdma = pltpu.make_async_remote_copy(
                    src_ref=comm.at[send_slot],
                    dst_ref=comm.at[recv_slot],
                    send_sem=send_sems.at[send_slot],
                    recv_sem=recv_sems.at[recv_slot],
                    device_id=(right,),
                    device_id_type=pl.DeviceIdType.MESH,
                )
                rdma.start()
                if k < N_DEV - 1:
                    b_in = (d - k - 1 + N_DEV) % N_DEV
                    acc[...] = partial(b_in, j)
                rdma.wait_send()
                if h <= total_hops - 2:
                    pl.semaphore_signal(credit_sem, inc=1, device_id=(left,),
                                        device_id_type=pl.DeviceIdType.MESH)
                rdma.wait_recv()
                if k < N_DEV - 2:
                    comm[recv_slot] = comm[recv_slot] + acc[...]
                elif k == N_DEV - 2:
                    comm[recv_slot] = jnp.maximum(
                        (comm[recv_slot] + acc[...]) * scale, 0.0)
                    store_out(recv_slot, (d + 1) % N_DEV, j)
                else:
                    t = k - (N_DEV - 1)
                    store_out(recv_slot, (d - t + N_DEV) % N_DEV, j)
                h += 1

    grid_spec = pltpu.PrefetchScalarGridSpec(
        num_scalar_prefetch=0,
        in_specs=[
            pl.BlockSpec(memory_space=pltpu.VMEM),
            pl.BlockSpec(memory_space=pltpu.VMEM),
            pl.BlockSpec(memory_space=pltpu.SMEM),
            pl.BlockSpec(memory_space=pltpu.SMEM),
        ],
        out_specs=pl.BlockSpec(memory_space=pltpu.ANY),
        scratch_shapes=[
            pltpu.VMEM((2, blk, n), jnp.float32),
            pltpu.VMEM((blk, n), jnp.float32),
            pltpu.SemaphoreType.DMA((2,)),
            pltpu.SemaphoreType.DMA((2,)),
            pltpu.SemaphoreType.REGULAR,
            pltpu.SemaphoreType.DMA,
        ],
    )
    return pl.pallas_call(
        body,
        grid_spec=grid_spec,
        out_shape=jax.ShapeDtypeStruct((m, n), jnp.float32),
        compiler_params=pltpu.CompilerParams(collective_id=0),
    )(x, w, scale_x, scale_w)


# baseline (device time: 2736869 ns/iter reference)
import jax
import jax.numpy as jnp
from jax import lax
from jax.experimental import pallas as pl
from jax.experimental.pallas import tpu as pltpu

N_DEV = 8
N_PASS = 2


def kernel(x, w_mat, scale_x, scale_w):
    x = x.astype(jnp.bfloat16)
    w = w_mat.astype(jnp.bfloat16)
    m, k_sh = x.shape
    _, n = w.shape
    ring_rows = m // N_DEV
    blk = ring_rows // N_PASS
    n_hops = 2 * (N_DEV - 1)
    total_hops = N_PASS * n_hops

    def body(x_ref, w_ref, sx_ref, sw_ref, out_ref,
             comm, acc, send_sems, recv_sems, credit_sem, out_sem):
        d = lax.axis_index("i")
        left = (d - 1 + N_DEV) % N_DEV
        right = (d + 1) % N_DEV
        scale = sx_ref[0] * sw_ref[0]

        barrier = pltpu.get_barrier_semaphore()
        for nbr in (left, right):
            pl.semaphore_signal(barrier, inc=1, device_id=(nbr,),
                                device_id_type=pl.DeviceIdType.MESH)
        pl.semaphore_wait(barrier, 2)

        def partial(block, j):
            rows = x_ref[pl.ds(block * ring_rows + j * blk, blk), :]
            return jnp.dot(rows, w_ref[...], preferred_element_type=jnp.float32)

        def store_out(slot, block, j):
            cp = pltpu.make_async_copy(
                comm.at[slot],
                out_ref.at[pl.ds(block * ring_rows + j * blk, blk), :],
                out_sem,
            )
            cp.start()
            cp.wait()

        h = 0
        for j in range(N_PASS):
            comm[0] = partial(d, j)
            for k in range(n_hops):
                send_slot = k % 2
                recv_slot = (k + 1) % 2
                if h >= 1:
                    pl.semaphore_wait(credit_sem, 1)
                rdma = pltpu.make_async_remote_copy(
                    src_ref=comm.at[send_slot],
                    dst_ref=comm.at[recv_slot],
                    send_sem=send_sems.at[send_slot],
                    recv_sem=recv_sems.at[recv_slot],
                    device_id=(right,),
                    device_id_type=pl.DeviceIdType.MESH,
                )
                rdma.start()
                if k < N_DEV - 1:
                    b_in = (d - k - 1 + N_DEV) % N_DEV
                    acc[...] = partial(b_in, j)
                rdma.wait_send()
                if h <= total_hops - 2:
                    pl.semaphore_signal(credit_sem, inc=1, device_id=(left,),
                                        device_id_type=pl.DeviceIdType.MESH)
                rdma.wait_recv()
                if k < N_DEV - 2:
                    comm[recv_slot] = comm[recv_slot] + acc[...]
                elif k == N_DEV - 2:
                    comm[recv_slot] = jnp.maximum(
                        (comm[recv_slot] + acc[...]) * scale, 0.0)
                    store_out(recv_slot, (d + 1) % N_DEV, j)
                else:
                    t = k - (N_DEV - 1)
                    store_out(recv_slot, (d - t + N_DEV) % N_DEV, j)
                h += 1

    grid_spec = pltpu.PrefetchScalarGridSpec(
        num_scalar_prefetch=0,
        in_specs=[
            pl.BlockSpec(memory_space=pltpu.VMEM),
            pl.BlockSpec(memory_space=pltpu.VMEM),
            pl.BlockSpec(memory_space=pltpu.SMEM),
            pl.BlockSpec(memory_space=pltpu.SMEM),
        ],
        out_specs=pl.BlockSpec(memory_space=pl.ANY),
        scratch_shapes=[
            pltpu.VMEM((2, blk, n), jnp.float32),
            pltpu.VMEM((blk, n), jnp.float32),
            pltpu.SemaphoreType.DMA((2,)),
            pltpu.SemaphoreType.DMA((2,)),
            pltpu.SemaphoreType.REGULAR,
            pltpu.SemaphoreType.DMA,
        ],
    )
    return pl.pallas_call(
        body,
        grid_spec=grid_spec,
        out_shape=jax.ShapeDtypeStruct((m, n), jnp.float32),
        compiler_params=pltpu.CompilerParams(collective_id=0),
    )(x, w, scale_x, scale_w)


# device time: 870776 ns/iter; 3.1430x vs baseline; 3.1430x over previous
import jax
import jax.numpy as jnp
from jax import lax
from jax.experimental import pallas as pl
from jax.experimental.pallas import tpu as pltpu

N_DEV = 8
N_PASS = 2


def kernel(x, w_mat, scale_x, scale_w):
    x = x.astype(jnp.bfloat16)
    w = w_mat.astype(jnp.bfloat16)
    m, k_sh = x.shape
    _, n = w.shape
    half = n // 2
    ring_rows = m // N_DEV
    blk = ring_rows // N_PASS
    n_hops = 2 * (N_DEV - 1)
    total_hops = N_PASS * n_hops

    def body(x_ref, w_ref, sx_ref, sw_ref, out_ref,
             comm_r, comm_l, acc_r, acc_l,
             sems_r, sems_l, recv_r, recv_l,
             credit_r, credit_l, out_sem_r, out_sem_l):
        d = lax.axis_index("i")
        left = (d - 1 + N_DEV) % N_DEV
        right = (d + 1) % N_DEV
        scale = sx_ref[0] * sw_ref[0]

        barrier = pltpu.get_barrier_semaphore()
        for nbr in (left, right):
            pl.semaphore_signal(barrier, inc=1, device_id=(nbr,),
                                device_id_type=pl.DeviceIdType.MESH)
        pl.semaphore_wait(barrier, 2)

        def partial(block, j, col0):
            rows = x_ref[pl.ds(block * ring_rows + j * blk, blk), :]
            return jnp.dot(rows, w_ref[:, col0:col0 + half],
                           preferred_element_type=jnp.float32)

        def store_out(stage, sem, block, j, col0):
            cp = pltpu.make_async_copy(
                stage,
                out_ref.at[pl.ds(block * ring_rows + j * blk, blk),
                           pl.ds(col0, half)],
                sem,
            )
            cp.start()
            cp.wait()

        h = 0
        for j in range(N_PASS):
            comm_r[0] = partial(d, j, 0).astype(jnp.bfloat16)
            comm_l[0] = partial(d, j, half).astype(jnp.bfloat16)
            for k in range(n_hops):
                ss = k % 2
                rs_ = (k + 1) % 2
                if h >= 1:
                    pl.semaphore_wait(credit_r, 1)
                    pl.semaphore_wait(credit_l, 1)
                rd_r = pltpu.make_async_remote_copy(
                    src_ref=comm_r.at[ss], dst_ref=comm_r.at[rs_],
                    send_sem=sems_r.at[ss], recv_sem=recv_r.at[rs_],
                    device_id=(right,), device_id_type=pl.DeviceIdType.MESH)
                rd_l = pltpu.make_async_remote_copy(
                    src_ref=comm_l.at[ss], dst_ref=comm_l.at[rs_],
                    send_sem=sems_l.at[ss], recv_sem=recv_l.at[rs_],
                    device_id=(left,), device_id_type=pl.DeviceIdType.MESH)
                rd_r.start()
                rd_l.start()
                if k < N_DEV - 1:
                    acc_r[...] = partial((d - k - 1 + N_DEV) % N_DEV, j, 0)
                    acc_l[...] = partial((d + k + 1) % N_DEV, j, half)
                rd_r.wait_send()
                rd_l.wait_send()
                if h <= total_hops - 2:
                    pl.semaphore_signal(credit_r, inc=1, device_id=(left,),
                                        device_id_type=pl.DeviceIdType.MESH)
                    pl.semaphore_signal(credit_l, inc=1, device_id=(right,),
                                        device_id_type=pl.DeviceIdType.MESH)
                rd_r.wait_recv()
                rd_l.wait_recv()
                if k < N_DEV - 2:
                    comm_r[rs_] = (comm_r[rs_].astype(jnp.float32)
                                   + acc_r[...]).astype(jnp.bfloat16)
                    comm_l[rs_] = (comm_l[rs_].astype(jnp.float32)
                                   + acc_l[...]).astype(jnp.bfloat16)
                elif k == N_DEV - 2:
                    acc_r[...] = jnp.maximum(
                        (comm_r[rs_].astype(jnp.float32) + acc_r[...]) * scale,
                        0.0)
                    acc_l[...] = jnp.maximum(
                        (comm_l[rs_].astype(jnp.float32) + acc_l[...]) * scale,
                        0.0)
                    comm_r[rs_] = acc_r[...].astype(jnp.bfloat16)
                    comm_l[rs_] = acc_l[...].astype(jnp.bfloat16)
                    store_out(acc_r, out_sem_r, (d + 1) % N_DEV, j, 0)
                    store_out(acc_l, out_sem_l, (d - 1 + N_DEV) % N_DEV, j,
                              half)
                else:
                    t = k - (N_DEV - 1)
                    acc_r[...] = comm_r[rs_].astype(jnp.float32)
                    acc_l[...] = comm_l[rs_].astype(jnp.float32)
                    store_out(acc_r, out_sem_r, (d - t + N_DEV) % N_DEV, j, 0)
                    store_out(acc_l, out_sem_l, (d + t) % N_DEV, j, half)
                h += 1

    grid_spec = pltpu.PrefetchScalarGridSpec(
        num_scalar_prefetch=0,
        in_specs=[
            pl.BlockSpec(memory_space=pltpu.VMEM),
            pl.BlockSpec(memory_space=pltpu.VMEM),
            pl.BlockSpec(memory_space=pltpu.SMEM),
            pl.BlockSpec(memory_space=pltpu.SMEM),
        ],
        out_specs=pl.BlockSpec(memory_space=pl.ANY),
        scratch_shapes=[
            pltpu.VMEM((2, blk, half), jnp.bfloat16),
            pltpu.VMEM((2, blk, half), jnp.bfloat16),
            pltpu.VMEM((blk, half), jnp.float32),
            pltpu.VMEM((blk, half), jnp.float32),
            pltpu.SemaphoreType.DMA((2,)),
            pltpu.SemaphoreType.DMA((2,)),
            pltpu.SemaphoreType.DMA((2,)),
            pltpu.SemaphoreType.DMA((2,)),
            pltpu.SemaphoreType.REGULAR,
            pltpu.SemaphoreType.REGULAR,
            pltpu.SemaphoreType.DMA,
            pltpu.SemaphoreType.DMA,
        ],
    )
    return pl.pallas_call(
        body,
        grid_spec=grid_spec,
        out_shape=jax.ShapeDtypeStruct((m, n), jnp.float32),
        compiler_params=pltpu.CompilerParams(collective_id=0),
    )(x, w, scale_x, scale_w)


# device time: 815594 ns/iter; 3.3557x vs baseline; 1.0677x over previous
import jax
import jax.numpy as jnp
from jax import lax
from jax.experimental import pallas as pl
from jax.experimental.pallas import tpu as pltpu

N_DEV = 8
N_PASS = 2


def kernel(x, w_mat, scale_x, scale_w):
    x = x.astype(jnp.bfloat16)
    w = w_mat.astype(jnp.bfloat16)
    m, k_sh = x.shape
    _, n = w.shape
    half = n // 2
    ring_rows = m // N_DEV
    blk = ring_rows // N_PASS
    n_hops = 2 * (N_DEV - 1)
    total_hops = N_PASS * n_hops

    def body(x_ref, w_ref, sx_ref, sw_ref, out_ref,
             comm_r, comm_l, acc_r, acc_l,
             sems_r, sems_l, recv_r, recv_l,
             credit_r, credit_l, out_sem_r, out_sem_l):
        d = lax.axis_index("i")
        left = (d - 1 + N_DEV) % N_DEV
        right = (d + 1) % N_DEV
        scale = sx_ref[0] * sw_ref[0]

        barrier = pltpu.get_barrier_semaphore()
        for nbr in (left, right):
            pl.semaphore_signal(barrier, inc=1, device_id=(nbr,),
                                device_id_type=pl.DeviceIdType.MESH)
        pl.semaphore_wait(barrier, 2)

        def partial(block, j, col0):
            rows = x_ref[pl.ds(block * ring_rows + j * blk, blk), :]
            return jnp.dot(rows, w_ref[:, col0:col0 + half],
                           preferred_element_type=jnp.float32)

        def store_out(stage, sem, block, j, col0):
            cp = pltpu.make_async_copy(
                stage,
                out_ref.at[pl.ds(block * ring_rows + j * blk, blk),
                           pl.ds(col0, half)],
                sem,
            )
            cp.start()
            cp.wait()

        h = 0
        for j in range(N_PASS):
            comm_r[0] = partial(d, j, 0).astype(jnp.bfloat16)
            comm_l[0] = partial(d, j, half).astype(jnp.bfloat16)
            pending = None
            for k in range(n_hops):
                ss = k % 2
                rs_ = (k + 1) % 2
                if h >= 1:
                    pl.semaphore_wait(credit_r, 1)
                    pl.semaphore_wait(credit_l, 1)
                rd_r = pltpu.make_async_remote_copy(
                    src_ref=comm_r.at[ss], dst_ref=comm_r.at[rs_],
                    send_sem=sems_r.at[ss], recv_sem=recv_r.at[rs_],
                    device_id=(right,), device_id_type=pl.DeviceIdType.MESH)
                rd_l = pltpu.make_async_remote_copy(
                    src_ref=comm_l.at[ss], dst_ref=comm_l.at[rs_],
                    send_sem=sems_l.at[ss], recv_sem=recv_l.at[rs_],
                    device_id=(left,), device_id_type=pl.DeviceIdType.MESH)
                rd_r.start()
                rd_l.start()
                if k < N_DEV - 1:
                    acc_r[...] = partial((d - k - 1 + N_DEV) % N_DEV, j, 0)
                    acc_l[...] = partial((d + k + 1) % N_DEV, j, half)
                elif pending is not None:
                    blk_r, blk_l, pslot, needs_cast = pending
                    if needs_cast:
                        acc_r[...] = comm_r[pslot].astype(jnp.float32)
                        acc_l[...] = comm_l[pslot].astype(jnp.float32)
                    store_out(acc_r, out_sem_r, blk_r, j, 0)
                    store_out(acc_l, out_sem_l, blk_l, j, half)
                    pending = None
                rd_r.wait_send()
                rd_l.wait_send()
                if h <= total_hops - 2:
                    pl.semaphore_signal(credit_r, inc=1, device_id=(left,),
                                        device_id_type=pl.DeviceIdType.MESH)
                    pl.semaphore_signal(credit_l, inc=1, device_id=(right,),
                                        device_id_type=pl.DeviceIdType.MESH)
                rd_r.wait_recv()
                rd_l.wait_recv()
                if k < N_DEV - 2:
                    comm_r[rs_] = (comm_r[rs_].astype(jnp.float32)
                                   + acc_r[...]).astype(jnp.bfloat16)
                    comm_l[rs_] = (comm_l[rs_].astype(jnp.float32)
                                   + acc_l[...]).astype(jnp.bfloat16)
                elif k == N_DEV - 2:
                    acc_r[...] = jnp.maximum(
                        (comm_r[rs_].astype(jnp.float32) + acc_r[...]) * scale,
                        0.0)
                    acc_l[...] = jnp.maximum(
                        (comm_l[rs_].astype(jnp.float32) + acc_l[...]) * scale,
                        0.0)
                    comm_r[rs_] = acc_r[...].astype(jnp.bfloat16)
                    comm_l[rs_] = acc_l[...].astype(jnp.bfloat16)
                    pending = ((d + 1) % N_DEV, (d - 1 + N_DEV) % N_DEV,
                               rs_, False)
                else:
                    t = k - (N_DEV - 1)
                    pending = ((d - t + N_DEV) % N_DEV, (d + t) % N_DEV,
                               rs_, True)
                h += 1
            blk_r, blk_l, pslot, needs_cast = pending
            acc_r[...] = comm_r[pslot].astype(jnp.float32)
            acc_l[...] = comm_l[pslot].astype(jnp.float32)
            store_out(acc_r, out_sem_r, blk_r, j, 0)
            store_out(acc_l, out_sem_l, blk_l, j, half)

    grid_spec = pltpu.PrefetchScalarGridSpec(
        num_scalar_prefetch=0,
        in_specs=[
            pl.BlockSpec(memory_space=pltpu.VMEM),
            pl.BlockSpec(memory_space=pltpu.VMEM),
            pl.BlockSpec(memory_space=pltpu.SMEM),
            pl.BlockSpec(memory_space=pltpu.SMEM),
        ],
        out_specs=pl.BlockSpec(memory_space=pl.ANY),
        scratch_shapes=[
            pltpu.VMEM((2, blk, half), jnp.bfloat16),
            pltpu.VMEM((2, blk, half), jnp.bfloat16),
            pltpu.VMEM((blk, half), jnp.float32),
            pltpu.VMEM((blk, half), jnp.float32),
            pltpu.SemaphoreType.DMA((2,)),
            pltpu.SemaphoreType.DMA((2,)),
            pltpu.SemaphoreType.DMA((2,)),
            pltpu.SemaphoreType.DMA((2,)),
            pltpu.SemaphoreType.REGULAR,
            pltpu.SemaphoreType.REGULAR,
            pltpu.SemaphoreType.DMA,
            pltpu.SemaphoreType.DMA,
        ],
    )
    return pl.pallas_call(
        body,
        grid_spec=grid_spec,
        out_shape=jax.ShapeDtypeStruct((m, n), jnp.float32),
        compiler_params=pltpu.CompilerParams(collective_id=0),
    )(x, w, scale_x, scale_w)


# device time: 784054 ns/iter; 3.4907x vs baseline; 1.0402x over previous
import jax
import jax.numpy as jnp
from jax import lax
from jax.experimental import pallas as pl
from jax.experimental.pallas import tpu as pltpu

N_DEV = 8
N_PASS = 1


def kernel(x, w_mat, scale_x, scale_w):
    x = x.astype(jnp.bfloat16)
    w = w_mat.astype(jnp.bfloat16)
    m, k_sh = x.shape
    _, n = w.shape
    half = n // 2
    ring_rows = m // N_DEV
    blk = ring_rows // N_PASS
    n_hops = 2 * (N_DEV - 1)
    total_hops = N_PASS * n_hops

    def body(x_ref, w_ref, sx_ref, sw_ref, out_ref,
             comm_r, comm_l, acc_r, acc_l,
             sems_r, sems_l, recv_r, recv_l,
             credit_r, credit_l, out_sem_r, out_sem_l):
        d = lax.axis_index("i")
        left = (d - 1 + N_DEV) % N_DEV
        right = (d + 1) % N_DEV
        scale = sx_ref[0] * sw_ref[0]

        barrier = pltpu.get_barrier_semaphore()
        for nbr in (left, right):
            pl.semaphore_signal(barrier, inc=1, device_id=(nbr,),
                                device_id_type=pl.DeviceIdType.MESH)
        pl.semaphore_wait(barrier, 2)

        def partial(block, j, col0):
            rows = x_ref[pl.ds(block * ring_rows + j * blk, blk), :]
            return jnp.dot(rows, w_ref[:, col0:col0 + half],
                           preferred_element_type=jnp.float32)

        def store_out(stage, sem, block, j, col0):
            cp = pltpu.make_async_copy(
                stage,
                out_ref.at[pl.ds(block * ring_rows + j * blk, blk),
                           pl.ds(col0, half)],
                sem,
            )
            cp.start()
            cp.wait()

        h = 0
        for j in range(N_PASS):
            comm_r[0] = partial(d, j, 0).astype(jnp.bfloat16)
            comm_l[0] = partial(d, j, half).astype(jnp.bfloat16)
            pending = None
            for k in range(n_hops):
                ss = k % 2
                rs_ = (k + 1) % 2
                if h >= 1:
                    pl.semaphore_wait(credit_r, 1)
                    pl.semaphore_wait(credit_l, 1)
                rd_r = pltpu.make_async_remote_copy(
                    src_ref=comm_r.at[ss], dst_ref=comm_r.at[rs_],
                    send_sem=sems_r.at[ss], recv_sem=recv_r.at[rs_],
                    device_id=(right,), device_id_type=pl.DeviceIdType.MESH)
                rd_l = pltpu.make_async_remote_copy(
                    src_ref=comm_l.at[ss], dst_ref=comm_l.at[rs_],
                    send_sem=sems_l.at[ss], recv_sem=recv_l.at[rs_],
                    device_id=(left,), device_id_type=pl.DeviceIdType.MESH)
                rd_r.start()
                rd_l.start()
                if k < N_DEV - 1:
                    acc_r[...] = partial((d - k - 1 + N_DEV) % N_DEV, j, 0)
                    acc_l[...] = partial((d + k + 1) % N_DEV, j, half)
                elif pending is not None:
                    blk_r, blk_l, pslot, needs_cast = pending
                    if needs_cast:
                        acc_r[...] = comm_r[pslot].astype(jnp.float32)
                        acc_l[...] = comm_l[pslot].astype(jnp.float32)
                    store_out(acc_r, out_sem_r, blk_r, j, 0)
                    store_out(acc_l, out_sem_l, blk_l, j, half)
                    pending = None
                rd_r.wait_send()
                rd_l.wait_send()
                if h <= total_hops - 2:
                    pl.semaphore_signal(credit_r, inc=1, device_id=(left,),
                                        device_id_type=pl.DeviceIdType.MESH)
                    pl.semaphore_signal(credit_l, inc=1, device_id=(right,),
                                        device_id_type=pl.DeviceIdType.MESH)
                rd_r.wait_recv()
                rd_l.wait_recv()
                if k < N_DEV - 2:
                    comm_r[rs_] = (comm_r[rs_].astype(jnp.float32)
                                   + acc_r[...]).astype(jnp.bfloat16)
                    comm_l[rs_] = (comm_l[rs_].astype(jnp.float32)
                                   + acc_l[...]).astype(jnp.bfloat16)
                elif k == N_DEV - 2:
                    acc_r[...] = jnp.maximum(
                        (comm_r[rs_].astype(jnp.float32) + acc_r[...]) * scale,
                        0.0)
                    acc_l[...] = jnp.maximum(
                        (comm_l[rs_].astype(jnp.float32) + acc_l[...]) * scale,
                        0.0)
                    comm_r[rs_] = acc_r[...].astype(jnp.bfloat16)
                    comm_l[rs_] = acc_l[...].astype(jnp.bfloat16)
                    pending = ((d + 1) % N_DEV, (d - 1 + N_DEV) % N_DEV,
                               rs_, False)
                else:
                    t = k - (N_DEV - 1)
                    pending = ((d - t + N_DEV) % N_DEV, (d + t) % N_DEV,
                               rs_, True)
                h += 1
            blk_r, blk_l, pslot, needs_cast = pending
            acc_r[...] = comm_r[pslot].astype(jnp.float32)
            acc_l[...] = comm_l[pslot].astype(jnp.float32)
            store_out(acc_r, out_sem_r, blk_r, j, 0)
            store_out(acc_l, out_sem_l, blk_l, j, half)

    grid_spec = pltpu.PrefetchScalarGridSpec(
        num_scalar_prefetch=0,
        in_specs=[
            pl.BlockSpec(memory_space=pltpu.VMEM),
            pl.BlockSpec(memory_space=pltpu.VMEM),
            pl.BlockSpec(memory_space=pltpu.SMEM),
            pl.BlockSpec(memory_space=pltpu.SMEM),
        ],
        out_specs=pl.BlockSpec(memory_space=pl.ANY),
        scratch_shapes=[
            pltpu.VMEM((2, blk, half), jnp.bfloat16),
            pltpu.VMEM((2, blk, half), jnp.bfloat16),
            pltpu.VMEM((blk, half), jnp.float32),
            pltpu.VMEM((blk, half), jnp.float32),
            pltpu.SemaphoreType.DMA((2,)),
            pltpu.SemaphoreType.DMA((2,)),
            pltpu.SemaphoreType.DMA((2,)),
            pltpu.SemaphoreType.DMA((2,)),
            pltpu.SemaphoreType.REGULAR,
            pltpu.SemaphoreType.REGULAR,
            pltpu.SemaphoreType.DMA,
            pltpu.SemaphoreType.DMA,
        ],
    )
    return pl.pallas_call(
        body,
        grid_spec=grid_spec,
        out_shape=jax.ShapeDtypeStruct((m, n), jnp.float32),
        compiler_params=pltpu.CompilerParams(collective_id=0),
    )(x, w, scale_x, scale_w)


# device time: 776140 ns/iter; 3.5263x vs baseline; 1.0102x over previous
import jax
import jax.numpy as jnp
from jax import lax
from jax.experimental import pallas as pl
from jax.experimental.pallas import tpu as pltpu

N_DEV = 8
N_PASS = 1


def _ring_pos(d):
    return jnp.where(d < 4, d, 11 - d)


def _ring_dev(r):
    return jnp.where(r < 4, r, 11 - r)


def kernel(x, w_mat, scale_x, scale_w):
    x = x.astype(jnp.bfloat16)
    w = w_mat.astype(jnp.bfloat16)
    m, k_sh = x.shape
    _, n = w.shape
    half = n // 2
    ring_rows = m // N_DEV
    blk = ring_rows // N_PASS
    n_hops = 2 * (N_DEV - 1)
    total_hops = N_PASS * n_hops

    def body(x_ref, w_ref, sx_ref, sw_ref, out_ref,
             comm_r, comm_l, acc_r, acc_l,
             sems_r, sems_l, recv_r, recv_l,
             credit_r, credit_l, out_sem_r, out_sem_l):
        d = lax.axis_index("i")
        r = _ring_pos(d)
        succ = _ring_dev((r + 1) % N_DEV)
        pred = _ring_dev((r + N_DEV - 1) % N_DEV)
        scale = sx_ref[0] * sw_ref[0]

        barrier = pltpu.get_barrier_semaphore()
        for nbr in (pred, succ):
            pl.semaphore_signal(barrier, inc=1, device_id=(nbr,),
                                device_id_type=pl.DeviceIdType.MESH)
        pl.semaphore_wait(barrier, 2)

        def partial(block, j, col0):
            rows = x_ref[pl.ds(block * ring_rows + j * blk, blk), :]
            return jnp.dot(rows, w_ref[:, col0:col0 + half],
                           preferred_element_type=jnp.float32)

        def store_out(stage, sem, block, j, col0):
            cp = pltpu.make_async_copy(
                stage,
                out_ref.at[pl.ds(block * ring_rows + j * blk, blk),
                           pl.ds(col0, half)],
                sem,
            )
            cp.start()
            cp.wait()

        def flush(pending, j):
            blk_r, blk_l, pslot = pending
            acc_r[...] = jnp.maximum(
                comm_r[pslot].astype(jnp.float32) * scale, 0.0)
            acc_l[...] = jnp.maximum(
                comm_l[pslot].astype(jnp.float32) * scale, 0.0)
            store_out(acc_r, out_sem_r, blk_r, j, 0)
            store_out(acc_l, out_sem_l, blk_l, j, half)

        h = 0
        for j in range(N_PASS):
            comm_r[0] = partial(r, j, 0).astype(jnp.bfloat16)
            comm_l[0] = partial(r, j, half).astype(jnp.bfloat16)
            pending = None
            for k in range(n_hops):
                ss = k % 2
                rs_ = (k + 1) % 2
                if h >= 1:
                    pl.semaphore_wait(credit_r, 1)
                    pl.semaphore_wait(credit_l, 1)
                rd_r = pltpu.make_async_remote_copy(
                    src_ref=comm_r.at[ss], dst_ref=comm_r.at[rs_],
                    send_sem=sems_r.at[ss], recv_sem=recv_r.at[rs_],
                    device_id=(succ,), device_id_type=pl.DeviceIdType.MESH)
                rd_l = pltpu.make_async_remote_copy(
                    src_ref=comm_l.at[ss], dst_ref=comm_l.at[rs_],
                    send_sem=sems_l.at[ss], recv_sem=recv_l.at[rs_],
                    device_id=(pred,), device_id_type=pl.DeviceIdType.MESH)
                rd_r.start()
                rd_l.start()
                if k < N_DEV - 1:
                    acc_r[...] = partial((r - k - 1 + N_DEV) % N_DEV, j, 0)
                    acc_l[...] = partial((r + k + 1) % N_DEV, j, half)
                elif pending is not None:
                    flush(pending, j)
                    pending = None
                rd_r.wait_send()
                rd_l.wait_send()
                if h <= total_hops - 2:
                    pl.semaphore_signal(credit_r, inc=1, device_id=(pred,),
                                        device_id_type=pl.DeviceIdType.MESH)
                    pl.semaphore_signal(credit_l, inc=1, device_id=(succ,),
                                        device_id_type=pl.DeviceIdType.MESH)
                rd_r.wait_recv()
                rd_l.wait_recv()
                if k < N_DEV - 1:
                    comm_r[rs_] = (comm_r[rs_].astype(jnp.float32)
                                   + acc_r[...]).astype(jnp.bfloat16)
                    comm_l[rs_] = (comm_l[rs_].astype(jnp.float32)
                                   + acc_l[...]).astype(jnp.bfloat16)
                    if k == N_DEV - 2:
                        pending = ((r + 1) % N_DEV,
                                   (r - 1 + N_DEV) % N_DEV, rs_)
                else:
                    t = k - (N_DEV - 1)
                    pending = ((r - t + N_DEV) % N_DEV,
                               (r + t) % N_DEV, rs_)
                h += 1
            flush(pending, j)

    grid_spec = pltpu.PrefetchScalarGridSpec(
        num_scalar_prefetch=0,
        in_specs=[
            pl.BlockSpec(memory_space=pltpu.VMEM),
            pl.BlockSpec(memory_space=pltpu.VMEM),
            pl.BlockSpec(memory_space=pltpu.SMEM),
            pl.BlockSpec(memory_space=pltpu.SMEM),
        ],
        out_specs=pl.BlockSpec(memory_space=pl.ANY),
        scratch_shapes=[
            pltpu.VMEM((2, blk, half), jnp.bfloat16),
            pltpu.VMEM((2, blk, half), jnp.bfloat16),
            pltpu.VMEM((blk, half), jnp.float32),
            pltpu.VMEM((blk, half), jnp.float32),
            pltpu.SemaphoreType.DMA((2,)),
            pltpu.SemaphoreType.DMA((2,)),
            pltpu.SemaphoreType.DMA((2,)),
            pltpu.SemaphoreType.DMA((2,)),
            pltpu.SemaphoreType.REGULAR,
            pltpu.SemaphoreType.REGULAR,
            pltpu.SemaphoreType.DMA,
            pltpu.SemaphoreType.DMA,
        ],
    )
    return pl.pallas_call(
        body,
        grid_spec=grid_spec,
        out_shape=jax.ShapeDtypeStruct((m, n), jnp.float32),
        compiler_params=pltpu.CompilerParams(collective_id=0),
    )(x, w, scale_x, scale_w)


# device time: 775705 ns/iter; 3.5282x vs baseline; 1.0006x over previous
import jax
import jax.numpy as jnp
from jax import lax
from jax.experimental import pallas as pl
from jax.experimental.pallas import tpu as pltpu

N_DEV = 8
N_PASS = 1
N_CHUNK = 2


def _ring_map(v):
    return jnp.where(v < 4, v, 11 - v)


def kernel(x, w_mat, scale_x, scale_w):
    x = x.astype(jnp.bfloat16)
    w = w_mat.astype(jnp.bfloat16)
    m, k_sh = x.shape
    _, n = w.shape
    n_pipe = 2 * N_CHUNK
    cw = n // n_pipe
    ring_rows = m // N_DEV
    blk = ring_rows // N_PASS
    n_hops = 2 * (N_DEV - 1)
    total_hops = N_PASS * n_hops

    def body(x_ref, w_ref, sx_ref, sw_ref, out_ref,
             comm0, comm1, comm2, comm3, acc0, acc1, acc2, acc3,
             send_sems, recv_sems, credits, out_sems):
        comm = [comm0, comm1, comm2, comm3]
        acc = [acc0, acc1, acc2, acc3]
        d = lax.axis_index("i")
        r = _ring_map(d)
        succ = _ring_map((r + 1) % N_DEV)
        pred = _ring_map((r + N_DEV - 1) % N_DEV)
        scale = sx_ref[0] * sw_ref[0]
        dest = [succ, succ, pred, pred]
        upstream = [pred, pred, succ, succ]
        col0 = [p * cw for p in range(n_pipe)]

        barrier = pltpu.get_barrier_semaphore()
        for nbr in (pred, succ):
            pl.semaphore_signal(barrier, inc=1, device_id=(nbr,),
                                device_id_type=pl.DeviceIdType.MESH)
        pl.semaphore_wait(barrier, 2)

        def partial(block, j, c0):
            rows = x_ref[pl.ds(block * ring_rows + j * blk, blk), :]
            return jnp.dot(rows, w_ref[:, c0:c0 + cw],
                           preferred_element_type=jnp.float32)

        def flush(pending, j):
            blk_fwd, blk_rev, pslot = pending
            for p in range(n_pipe):
                block = blk_fwd if p < N_CHUNK else blk_rev
                acc[p][...] = jnp.maximum(
                    comm[p][pslot].astype(jnp.float32) * scale, 0.0)
                cp = pltpu.make_async_copy(
                    acc[p],
                    out_ref.at[pl.ds(block * ring_rows + j * blk, blk),
                               pl.ds(col0[p], cw)],
                    out_sems.at[p],
                )
                cp.start()
                cp.wait()

        h = 0
        for j in range(N_PASS):
            for p in range(n_pipe):
                comm[p][0] = partial(r, j, col0[p]).astype(jnp.bfloat16)
            pending = None
            for k in range(n_hops):
                ss = k % 2
                rs_ = (k + 1) % 2
                rd = []
                for p in range(n_pipe):
                    if h >= 1:
                        pl.semaphore_wait(credits.at[p], 1)
                    rd_p = pltpu.make_async_remote_copy(
                        src_ref=comm[p].at[ss], dst_ref=comm[p].at[rs_],
                        send_sem=send_sems.at[p, ss],
                        recv_sem=recv_sems.at[p, rs_],
                        device_id=(dest[p],),
                        device_id_type=pl.DeviceIdType.MESH)
                    rd_p.start()
                    rd.append(rd_p)
                if k < N_DEV - 1:
                    b_fwd = (r - k - 1 + N_DEV) % N_DEV
                    b_rev = (r + k + 1) % N_DEV
                    for p in range(n_pipe):
                        acc[p][...] = partial(
                            b_fwd if p < N_CHUNK else b_rev, j, col0[p])
                elif pending is not None:
                    flush(pending, j)
                    pending = None
                for p in range(n_pipe):
                    rd[p].wait_send()
                if h <= total_hops - 2:
                    for p in range(n_pipe):
                        pl.semaphore_signal(
                            credits.at[p], inc=1, device_id=(upstream[p],),
                            device_id_type=pl.DeviceIdType.MESH)
                for p in range(n_pipe):
                    rd[p].wait_recv()
                    if k < N_DEV - 1:
                        comm[p][rs_] = (comm[p][rs_].astype(jnp.float32)
                                        + acc[p][...]).astype(jnp.bfloat16)
                if k < N_DEV - 1:
                    if k == N_DEV - 2:
                        pending = ((r + 1) % N_DEV,
                                   (r - 1 + N_DEV) % N_DEV, rs_)
                else:
                    t = k - (N_DEV - 1)
                    pending = ((r - t + N_DEV) % N_DEV,
                               (r + t) % N_DEV, rs_)
                h += 1
            flush(pending, j)

    grid_spec = pltpu.PrefetchScalarGridSpec(
        num_scalar_prefetch=0,
        in_specs=[
            pl.BlockSpec(memory_space=pltpu.VMEM),
            pl.BlockSpec(memory_space=pltpu.VMEM),
            pl.BlockSpec(memory_space=pltpu.SMEM),
            pl.BlockSpec(memory_space=pltpu.SMEM),
        ],
        out_specs=pl.BlockSpec(memory_space=pl.ANY),
        scratch_shapes=(
            [pltpu.VMEM((2, blk, cw), jnp.bfloat16) for _ in range(4)]
            + [pltpu.VMEM((blk, cw), jnp.float32) for _ in range(4)]
            + [
                pltpu.SemaphoreType.DMA((4, 2)),
                pltpu.SemaphoreType.DMA((4, 2)),
                pltpu.SemaphoreType.REGULAR((4,)),
                pltpu.SemaphoreType.DMA((4,)),
            ]
        ),
    )
    return pl.pallas_call(
        body,
        grid_spec=grid_spec,
        out_shape=jax.ShapeDtypeStruct((m, n), jnp.float32),
        compiler_params=pltpu.CompilerParams(collective_id=0),
    )(x, w, scale_x, scale_w)
